# baseline (speedup 1.0000x reference)
import jax
import jax.numpy as jnp
from jax.experimental import pallas as pl
from jax.experimental.pallas import tpu as pltpu

B, T, N, F, GF = 512, 12, 200, 16, 1
GH, OUT, FCH, NL = 64, 512, 512, 3
EPS = 1e-5
HB = B // 2


def _bn_cols(h, g, be):
    m = jnp.mean(h, axis=0, keepdims=True)
    v = jnp.mean((h - m) * (h - m), axis=0, keepdims=True)
    return g * (h - m) * jax.lax.rsqrt(v + EPS) + be


def _stgcn_body(x1t_ref, x2_ref, cheb1_ref, wg_ref, bg_ref,
                wx_ref, wh_ref, bl_ref, wout_ref, bout_ref,
                w0_ref, b0_ref, g0_ref, be0_ref,
                g1_ref, be1_ref, w1_ref, b1_ref,
                o_ref, seq):
    sid = x2_ref[:, F - 1:F].astype(jnp.int32)
    ids = jax.lax.broadcasted_iota(jnp.int32, (B, N), 1)
    onehot = (ids == sid).astype(jnp.float32)
    rows1 = jnp.dot(onehot, cheb1_ref[...],
                    preferred_element_type=jnp.float32)

    wg0b = jnp.broadcast_to(wg_ref[0], (N, GH))
    wg1b = jnp.broadcast_to(wg_ref[1], (N, GH))
    bg = bg_ref[...]
    for t in range(T):
        xt = x1t_ref[t]
        xg = jnp.maximum(
            jnp.dot(onehot * xt, wg0b, preferred_element_type=jnp.float32)
            + jnp.dot(rows1 * xt, wg1b, preferred_element_type=jnp.float32)
            + bg, 0.0)
        seq[t * B:(t + 1) * B, :] = xg

    lane = jax.lax.broadcasted_iota(jnp.int32, (1, 4 * GH), 1)
    gs = jnp.where((lane < 2 * GH) | (lane >= 3 * GH), 0.5, 1.0)

    def _step(x_in, h, c, wcat, blg):
        z = jnp.dot(jnp.concatenate([x_in, h], axis=1), wcat,
                    preferred_element_type=jnp.float32) + blg
        ti = jnp.tanh(z[:, :GH])
        tf = jnp.tanh(z[:, GH:2 * GH])
        tg = jnp.tanh(z[:, 2 * GH:3 * GH])
        to = jnp.tanh(z[:, 3 * GH:])
        c = 0.5 * (tf * c + c + ti * tg + tg)
        tc = jnp.tanh(c)
        return to * tc + tc, c

    QB = B // 4
    hs = [jnp.zeros((QB, GH), jnp.float32)] * 4
    for l in range(NL):
        wx_s = wx_ref[l] * (gs if l == 0 else 0.5 * gs)
        wh_s = wh_ref[l] * (0.5 * gs)
        wcat = jnp.concatenate([wx_s, wh_s], axis=0)
        blg = bl_ref[l:l + 1, :] * gs
        hs = [jnp.zeros((QB, GH), jnp.float32)] * 4
        cs = [jnp.zeros((QB, GH), jnp.float32)] * 4
        for t in range(T):
            base = t * B
            for q in range(4):
                hs[q], cs[q] = _step(seq[base + q * QB:base + (q + 1) * QB, :],
                                     hs[q], cs[q], wcat, blg)
            for q in range(4):
                seq[base + q * QB:base + (q + 1) * QB, :] = hs[q]

    wout_s = wout_ref[...] * 0.5
    gF = jnp.concatenate(
        [jnp.dot(h, wout_s, preferred_element_type=jnp.float32) for h in hs],
        axis=0) + bout_ref[...]

    h2p = jnp.dot(x2_ref[:, :F - 1], w0_ref[...],
                  preferred_element_type=jnp.float32) + b0_ref[...]
    h2n = _bn_cols(h2p, g0_ref[...], be0_ref[...])
    h2 = jnp.where(h2n >= 0, h2n, 0.01 * h2n)
    bna = _bn_cols(h2, g1_ref[:, :FCH], be1_ref[:, :FCH])
    bnb = _bn_cols(gF, g1_ref[:, FCH:], be1_ref[:, FCH:])
    y = (jnp.sum(bna * w1_ref[:, :FCH], axis=1, keepdims=True)
         + jnp.sum(bnb * w1_ref[:, FCH:], axis=1, keepdims=True)
         + b1_ref[...])
    o_ref[...] = 0.5 * jnp.tanh(0.5 * y) + 0.5


_CALL = pl.pallas_call(
    _stgcn_body,
    out_shape=jax.ShapeDtypeStruct((B, 1), jnp.float32),
    scratch_shapes=[
        pltpu.VMEM((T * B, GH), jnp.float32),
    ],
    compiler_params=pltpu.CompilerParams(
        vmem_limit_bytes=100 * 1024 * 1024,
        allow_input_fusion=(True, True) + (False,) * 16,
    ),
)


def kernel(x, cheb, W_g, b_g, Wx, Wh, b_lstm, W_out, b_out, W0, b0, g0, be0,
           g1, be1, W1, b1):
    x1t = x[:, :T * N * GF].reshape(B, T, N).transpose(1, 0, 2)
    x2 = x[:, T * N * GF:]
    return _CALL(x1t, x2, cheb[1], W_g, b_g.reshape(1, GH), Wx, Wh, b_lstm,
                 W_out, b_out.reshape(1, OUT),
                 W0, b0.reshape(1, FCH), g0.reshape(1, FCH),
                 be0.reshape(1, FCH),
                 g1.reshape(1, FCH + OUT), be1.reshape(1, FCH + OUT),
                 W1.reshape(1, FCH + OUT), b1.reshape(1, 1))


# speedup vs baseline: 1.0030x; 1.0030x over previous
import jax
import jax.numpy as jnp
from jax.experimental import pallas as pl
from jax.experimental.pallas import tpu as pltpu

B, T, N, F, GF = 512, 12, 200, 16, 1
GH, OUT, FCH, NL = 64, 512, 512, 3
EPS = 1e-5
HB = B // 2


def _bn_cols(h, g, be):
    m = jnp.mean(h, axis=0, keepdims=True)
    v = jnp.mean((h - m) * (h - m), axis=0, keepdims=True)
    return g * (h - m) * jax.lax.rsqrt(v + EPS) + be


def _stgcn_body(x1t_ref, x2_ref, cheb1_ref, wg_ref, bg_ref,
                wx_ref, wh_ref, bl_ref, wout_ref, bout_ref,
                w0_ref, b0_ref, g0_ref, be0_ref,
                g1_ref, be1_ref, w1_ref, b1_ref,
                o_ref, seq):
    sid = x2_ref[:, F - 1:F].astype(jnp.int32)
    ids = jax.lax.broadcasted_iota(jnp.int32, (B, N), 1)
    onehot = (ids == sid).astype(jnp.float32)
    rows1 = jnp.dot(onehot, cheb1_ref[...],
                    preferred_element_type=jnp.float32)

    wg0b = jnp.broadcast_to(wg_ref[0], (N, GH))
    wg1b = jnp.broadcast_to(wg_ref[1], (N, GH))
    bg = bg_ref[...]
    for t in range(T):
        xt = x1t_ref[t]
        xg = jnp.maximum(
            jnp.dot(onehot * xt, wg0b, preferred_element_type=jnp.float32)
            + jnp.dot(rows1 * xt, wg1b, preferred_element_type=jnp.float32)
            + bg, 0.0)
        seq[t * B:(t + 1) * B, :] = xg

    lane = jax.lax.broadcasted_iota(jnp.int32, (1, 4 * GH), 1)
    gs = jnp.where((lane < 2 * GH) | (lane >= 3 * GH), 0.5, 1.0)

    def _step(x_in, h, c, wcat, blg):
        z = jnp.dot(jnp.concatenate([x_in, h], axis=1), wcat,
                    preferred_element_type=jnp.float32) + blg
        ti = jnp.tanh(z[:, :GH])
        tf = jnp.tanh(z[:, GH:2 * GH])
        tg = jnp.tanh(z[:, 2 * GH:3 * GH])
        to = jnp.tanh(z[:, 3 * GH:])
        c = 0.5 * (tf * c + c + ti * tg + tg)
        tc = jnp.tanh(c)
        return to * tc + tc, c

    hA = jnp.zeros((HB, GH), jnp.float32)
    hB = jnp.zeros((HB, GH), jnp.float32)
    for l in range(NL):
        wx_s = wx_ref[l] * (gs if l == 0 else 0.5 * gs)
        wh_s = wh_ref[l] * (0.5 * gs)
        wcat = jnp.concatenate([wx_s, wh_s], axis=0)
        blg = bl_ref[l:l + 1, :] * gs
        hA = jnp.zeros((HB, GH), jnp.float32)
        cA = jnp.zeros((HB, GH), jnp.float32)
        hB = jnp.zeros((HB, GH), jnp.float32)
        cB = jnp.zeros((HB, GH), jnp.float32)
        for t in range(T):
            base = t * B
            hA, cA = _step(seq[base:base + HB, :], hA, cA, wcat, blg)
            hB, cB = _step(seq[base + HB:base + B, :], hB, cB, wcat, blg)
            seq[base:base + HB, :] = hA
            seq[base + HB:base + B, :] = hB

    wout_s = wout_ref[...] * 0.5
    gF = jnp.concatenate([
        jnp.dot(hA, wout_s, preferred_element_type=jnp.float32),
        jnp.dot(hB, wout_s, preferred_element_type=jnp.float32),
    ], axis=0) + bout_ref[...]

    h2p = jnp.dot(x2_ref[:, :F - 1], w0_ref[...],
                  preferred_element_type=jnp.float32) + b0_ref[...]
    h2n = _bn_cols(h2p, g0_ref[...], be0_ref[...])
    h2 = jnp.where(h2n >= 0, h2n, 0.01 * h2n)
    bna = _bn_cols(h2, g1_ref[:, :FCH], be1_ref[:, :FCH])
    bnb = _bn_cols(gF, g1_ref[:, FCH:], be1_ref[:, FCH:])
    y = (jnp.sum(bna * w1_ref[:, :FCH], axis=1, keepdims=True)
         + jnp.sum(bnb * w1_ref[:, FCH:], axis=1, keepdims=True)
         + b1_ref[...])
    o_ref[...] = 0.5 * jnp.tanh(0.5 * y) + 0.5


_CALL = pl.pallas_call(
    _stgcn_body,
    out_shape=jax.ShapeDtypeStruct((B, 1), jnp.float32),
    scratch_shapes=[
        pltpu.VMEM((T * B, GH), jnp.float32),
    ],
    compiler_params=pltpu.CompilerParams(
        vmem_limit_bytes=100 * 1024 * 1024,
        allow_input_fusion=(True, True) + (False,) * 16,
    ),
)


def kernel(x, cheb, W_g, b_g, Wx, Wh, b_lstm, W_out, b_out, W0, b0, g0, be0,
           g1, be1, W1, b1):
    x1t = x[:, :T * N * GF].reshape(B, T, N).transpose(1, 0, 2)
    x2 = x[:, T * N * GF:]
    return _CALL(x1t, x2, cheb[1], W_g, b_g.reshape(1, GH), Wx, Wh, b_lstm,
                 W_out, b_out.reshape(1, OUT),
                 W0, b0.reshape(1, FCH), g0.reshape(1, FCH),
                 be0.reshape(1, FCH),
                 g1.reshape(1, FCH + OUT), be1.reshape(1, FCH + OUT),
                 W1.reshape(1, FCH + OUT), b1.reshape(1, 1))
